# baseline (speedup 1.0000x reference)
import functools

import jax
import jax.numpy as jnp
from jax import lax
from jax.experimental import pallas as pl
from jax.experimental.pallas import tpu as pltpu
from jax.experimental.pallas import tpu_sc as plsc

M, G, DZ = 16, 16384, 128
NT, DY = 8192, 64
B = M * NT

NC, NS = 2, 16
NW = NC * NS
B_PER_W = B // NW
W_PER_ROW = NT // B_PER_W
CH = 128
NCHUNK = B_PER_W // CH
NBUF = 2


def _sc_gather():
    mesh = plsc.VectorSubcoreMesh(core_axis_name="c", subcore_axis_name="s")

    @functools.partial(
        pl.kernel,
        mesh=mesh,
        out_type=jax.ShapeDtypeStruct((M, NT, DZ), jnp.float32),
        scratch_types=[
            pltpu.VMEM((B_PER_W,), jnp.int32),
            *[pltpu.VMEM((CH, DZ), jnp.float32) for _ in range(NBUF)],
            *[pltpu.SemaphoreType.DMA for _ in range(2 * NBUF)],
        ],
    )
    def gather(table_hbm, mt_hbm, out_hbm, idx_v, *bufs_and_sems):
        rows = bufs_and_sems[:NBUF]
        gsem = bufs_and_sems[NBUF : 2 * NBUF]
        ssem = bufs_and_sems[2 * NBUF :]
        wid = lax.axis_index("s") * NC + lax.axis_index("c")
        i = wid // W_PER_ROW
        h = wid % W_PER_ROW
        col0 = h * B_PER_W
        pltpu.sync_copy(mt_hbm.at[i, pl.ds(col0, B_PER_W)], idx_v)

        def gather_chunk(c, b):
            return pltpu.async_copy(
                table_hbm.at[i].at[idx_v.at[pl.ds(c * CH, CH)]], rows[b], gsem[b]
            )

        gcp = [None] * NBUF
        scp = [None] * NBUF
        gcp[0] = gather_chunk(0, 0)
        for c in range(NCHUNK):
            b = c % NBUF
            nb = (c + 1) % NBUF
            if c + 1 < NCHUNK:
                if scp[nb] is not None:
                    scp[nb].wait()
                gcp[nb] = gather_chunk(c + 1, nb)
            gcp[b].wait()
            scp[b] = pltpu.async_copy(
                rows[b], out_hbm.at[i, pl.ds(col0 + c * CH, CH)], ssem[b]
            )
        for b in range(NBUF):
            if scp[b] is not None:
                scp[b].wait()

    return gather


_gather_fn = _sc_gather()


def _mm_body(zt_ref, w_ref, b_ref, o_ref):
    o_ref[0] = (
        jnp.dot(zt_ref[0], w_ref[...], preferred_element_type=jnp.float32)
        + b_ref[...]
    )


def _tc_linear(zt2, W2, b2):
    BN = 1024
    return pl.pallas_call(
        _mm_body,
        grid=(M, NT // 2 // BN),
        in_specs=[
            pl.BlockSpec((1, BN, 2 * DZ), lambda i, j: (i, j, 0)),
            pl.BlockSpec((2 * DZ, 2 * DY), lambda i, j: (0, 0)),
            pl.BlockSpec((1, 2 * DY), lambda i, j: (0, 0)),
        ],
        out_specs=pl.BlockSpec((1, BN, 2 * DY), lambda i, j: (i, j, 0)),
        out_shape=jax.ShapeDtypeStruct((M, NT // 2, 2 * DY), jnp.float32),
    )(zt2, W2, b2)


@jax.jit
def kernel(z_grid, mt, W, b):
    zt = _gather_fn(z_grid, mt.astype(jnp.int32))
    zt2 = zt.reshape(M, NT // 2, 2 * DZ)
    W2 = jnp.zeros((2 * DZ, 2 * DY), jnp.float32)
    W2 = W2.at[:DZ, :DY].set(W).at[DZ:, DY:].set(W)
    b2 = jnp.concatenate([b, b]).reshape(1, 2 * DY)
    out2 = _tc_linear(zt2, W2, b2)
    return out2.reshape(M, NT, DY)


# speedup vs baseline: 2.1444x; 2.1444x over previous
import functools

import jax
import jax.numpy as jnp
from jax import lax
from jax.experimental import pallas as pl
from jax.experimental.pallas import tpu as pltpu
from jax.experimental.pallas import tpu_sc as plsc

M, G, DZ = 16, 16384, 128
NT, DY = 8192, 64
B = M * NT

NC, NS = 2, 16
NW = NC * NS
B_PER_W = B // NW
W_PER_ROW = NT // B_PER_W
CH = 128
NCHUNK = B_PER_W // CH
NBUF = 2


def _sc_gather():
    mesh = plsc.VectorSubcoreMesh(core_axis_name="c", subcore_axis_name="s")

    @functools.partial(
        pl.kernel,
        mesh=mesh,
        out_type=jax.ShapeDtypeStruct((M, NT, DZ), jnp.float32),
        scratch_types=[
            pltpu.VMEM((B_PER_W,), jnp.int32),
            *[pltpu.VMEM((CH, DZ), jnp.float32) for _ in range(NBUF)],
            *[pltpu.SemaphoreType.DMA for _ in range(2 * NBUF)],
        ],
    )
    def gather(table_hbm, mt_hbm, out_hbm, idx_v, *bufs_and_sems):
        rows = bufs_and_sems[:NBUF]
        gsem = bufs_and_sems[NBUF : 2 * NBUF]
        ssem = bufs_and_sems[2 * NBUF :]
        wid = lax.axis_index("s") * NC + lax.axis_index("c")
        i = wid // W_PER_ROW
        h = wid % W_PER_ROW
        col0 = h * B_PER_W
        pltpu.sync_copy(mt_hbm.at[i, pl.ds(col0, B_PER_W)], idx_v)

        def gather_chunk(c, b):
            return pltpu.async_copy(
                table_hbm.at[i].at[idx_v.at[pl.ds(c * CH, CH)]], rows[b], gsem[b]
            )

        gcp = [None] * NBUF
        scp = [None] * NBUF
        gcp[0] = gather_chunk(0, 0)
        for c in range(NCHUNK):
            b = c % NBUF
            nb = (c + 1) % NBUF
            if c + 1 < NCHUNK:
                if scp[nb] is not None:
                    scp[nb].wait()
                gcp[nb] = gather_chunk(c + 1, nb)
            gcp[b].wait()
            scp[b] = pltpu.async_copy(
                rows[b], out_hbm.at[i, pl.ds(col0 + c * CH, CH)], ssem[b]
            )
        for b in range(NBUF):
            if scp[b] is not None:
                scp[b].wait()

    return gather


_gather_fn = _sc_gather()


def _mm_body(zt_ref, wt_ref, b_ref, o_ref):
    o_ref[0] = (
        jax.lax.dot_general(
            wt_ref[...],
            zt_ref[0],
            (((1,), (1,)), ((), ())),
            preferred_element_type=jnp.float32,
        )
        + b_ref[...]
    )


def _tc_linear(zt, Wt, b2):
    BN = 2048
    return pl.pallas_call(
        _mm_body,
        grid=(M, NT // BN),
        in_specs=[
            pl.BlockSpec((1, BN, DZ), lambda i, j: (i, j, 0)),
            pl.BlockSpec((DY, DZ), lambda i, j: (0, 0)),
            pl.BlockSpec((DY, 1), lambda i, j: (0, 0)),
        ],
        out_specs=pl.BlockSpec((1, DY, BN), lambda i, j: (i, 0, j)),
        out_shape=jax.ShapeDtypeStruct((M, DY, NT), jnp.float32),
    )(zt, Wt, b2)


@jax.jit
def kernel(z_grid, mt, W, b):
    zt = _gather_fn(z_grid, mt.astype(jnp.int32))
    out_t = _tc_linear(zt, W.T, b.reshape(DY, 1))
    return jnp.transpose(out_t, (0, 2, 1))


# speedup vs baseline: 2.3290x; 1.0861x over previous
import functools

import jax
import jax.numpy as jnp
from jax import lax
from jax.experimental import pallas as pl
from jax.experimental.pallas import tpu as pltpu
from jax.experimental.pallas import tpu_sc as plsc

M, G, DZ = 16, 16384, 128
NT, DY = 8192, 64
B = M * NT

NC, NS = 2, 16
NW = NC * NS
CH = 128
NBUF = 2

NSLICE = 4
MS = M // NSLICE
B_PER_W = MS * NT // NW
W_PER_ROW = NT // B_PER_W
NCHUNK = B_PER_W // CH


def _sc_gather(slice_idx):
    mesh = plsc.VectorSubcoreMesh(core_axis_name="c", subcore_axis_name="s")

    @functools.partial(
        pl.kernel,
        mesh=mesh,
        out_type=jax.ShapeDtypeStruct((MS, NT, DZ), jnp.float32),
        scratch_types=[
            pltpu.VMEM((B_PER_W,), jnp.int32),
            *[pltpu.VMEM((CH, DZ), jnp.float32) for _ in range(NBUF)],
            *[pltpu.SemaphoreType.DMA for _ in range(2 * NBUF)],
        ],
    )
    def gather(table_hbm, mt_hbm, out_hbm, idx_v, *bufs_and_sems):
        rows = bufs_and_sems[:NBUF]
        gsem = bufs_and_sems[NBUF : 2 * NBUF]
        ssem = bufs_and_sems[2 * NBUF :]
        wid = lax.axis_index("s") * NC + lax.axis_index("c")
        iloc = wid // W_PER_ROW
        i = slice_idx * MS + iloc
        h = wid % W_PER_ROW
        col0 = h * B_PER_W
        pltpu.sync_copy(mt_hbm.at[i, pl.ds(col0, B_PER_W)], idx_v)

        def gather_chunk(c, b):
            return pltpu.async_copy(
                table_hbm.at[i].at[idx_v.at[pl.ds(c * CH, CH)]], rows[b], gsem[b]
            )

        gcp = [None] * NBUF
        scp = [None] * NBUF
        gcp[0] = gather_chunk(0, 0)
        for c in range(NCHUNK):
            b = c % NBUF
            nb = (c + 1) % NBUF
            if c + 1 < NCHUNK:
                if scp[nb] is not None:
                    scp[nb].wait()
                gcp[nb] = gather_chunk(c + 1, nb)
            gcp[b].wait()
            scp[b] = pltpu.async_copy(
                rows[b], out_hbm.at[iloc, pl.ds(col0 + c * CH, CH)], ssem[b]
            )
        for b in range(NBUF):
            if scp[b] is not None:
                scp[b].wait()

    return gather


_gather_fns = [_sc_gather(s) for s in range(NSLICE)]


def _mm_body(zt_ref, wt_ref, b_ref, o_ref):
    o_ref[0] = (
        lax.dot_general(
            wt_ref[...],
            zt_ref[0],
            (((1,), (1,)), ((), ())),
            preferred_element_type=jnp.float32,
        )
        + b_ref[...]
    )


def _tc_linear_slice(slice_idx, zt_s, Wt, b2, out_prev):
    BN = 2048
    args = [zt_s, Wt, b2]
    in_specs = [
        pl.BlockSpec((1, BN, DZ), lambda i, j: (i, j, 0)),
        pl.BlockSpec((DY, DZ), lambda i, j: (0, 0)),
        pl.BlockSpec((DY, 1), lambda i, j: (0, 0)),
    ]
    aliases = {}
    if out_prev is not None:
        args.append(out_prev)
        in_specs.append(pl.BlockSpec(memory_space=pl.ANY))
        aliases = {3: 0}

    def body(zt_ref, wt_ref, b_ref, *rest):
        _mm_body(zt_ref, wt_ref, b_ref, rest[-1])

    return pl.pallas_call(
        body,
        grid=(MS, NT // BN),
        in_specs=in_specs,
        out_specs=pl.BlockSpec(
            (1, DY, BN), lambda i, j: (slice_idx * MS + i, 0, j)
        ),
        out_shape=jax.ShapeDtypeStruct((M, DY, NT), jnp.float32),
        input_output_aliases=aliases,
    )(*args)


@jax.jit
def kernel(z_grid, mt, W, b):
    mt32 = mt.astype(jnp.int32)
    Wt = W.T
    b2 = b.reshape(DY, 1)
    zts = [_gather_fns[s](z_grid, mt32) for s in range(NSLICE)]
    out_t = None
    for s in range(NSLICE):
        out_t = _tc_linear_slice(s, zts[s], Wt, b2, out_t)
    return jnp.transpose(out_t, (0, 2, 1))


# speedup vs baseline: 2.5043x; 1.0752x over previous
import functools

import jax
import jax.numpy as jnp
from jax import lax
from jax.experimental import pallas as pl
from jax.experimental.pallas import tpu as pltpu
from jax.experimental.pallas import tpu_sc as plsc

M, G, DZ = 16, 16384, 128
NT, DY = 8192, 64
B = M * NT

NC, NS = 2, 16
NW = NC * NS
CH = 128
NBUF = 2

NSLICE = 4
MS = M // NSLICE
B_PER_W = MS * NT // NW
W_PER_ROW = NT // B_PER_W
NCHUNK = B_PER_W // CH


def _sc_gather(slice_idx):
    mesh = plsc.VectorSubcoreMesh(core_axis_name="c", subcore_axis_name="s")

    @functools.partial(
        pl.kernel,
        mesh=mesh,
        out_type=jax.ShapeDtypeStruct((MS, NT, DZ), jnp.float32),
        scratch_types=[
            pltpu.VMEM((B_PER_W,), jnp.int32),
            *[pltpu.VMEM((CH, DZ), jnp.float32) for _ in range(NBUF)],
            *[pltpu.SemaphoreType.DMA for _ in range(2 * NBUF)],
        ],
    )
    def gather(table_hbm, mt_hbm, out_hbm, idx_v, *bufs_and_sems):
        rows = bufs_and_sems[:NBUF]
        gsem = bufs_and_sems[NBUF : 2 * NBUF]
        ssem = bufs_and_sems[2 * NBUF :]
        wid = lax.axis_index("s") * NC + lax.axis_index("c")
        iloc = wid // W_PER_ROW
        i = slice_idx * MS + iloc
        h = wid % W_PER_ROW
        col0 = h * B_PER_W
        pltpu.sync_copy(mt_hbm.at[i, pl.ds(col0, B_PER_W)], idx_v)

        def gather_chunk(c, b):
            return pltpu.async_copy(
                table_hbm.at[i].at[idx_v.at[pl.ds(c * CH, CH)]], rows[b], gsem[b]
            )

        gcp = [None] * NBUF
        scp = [None] * NBUF
        gcp[0] = gather_chunk(0, 0)
        for c in range(NCHUNK):
            b = c % NBUF
            nb = (c + 1) % NBUF
            if c + 1 < NCHUNK:
                if scp[nb] is not None:
                    scp[nb].wait()
                gcp[nb] = gather_chunk(c + 1, nb)
            gcp[b].wait()
            scp[b] = pltpu.async_copy(
                rows[b], out_hbm.at[iloc, pl.ds(col0 + c * CH, CH)], ssem[b]
            )
        for b in range(NBUF):
            if scp[b] is not None:
                scp[b].wait()

    return gather


_gather_fns = [_sc_gather(s) for s in range(NSLICE)]


def _mm_body(zt_ref, wt_ref, b_ref, o_ref):
    o_ref[0] = (
        lax.dot_general(
            wt_ref[...],
            zt_ref[0],
            (((1,), (1,)), ((), ())),
            preferred_element_type=jnp.float32,
        )
        + b_ref[...]
    )


def _tc_linear_slice(slice_idx, zt_s, Wt, b2, out_prev):
    BN = 4096
    args = [zt_s, Wt, b2]
    in_specs = [
        pl.BlockSpec((1, BN, DZ), lambda i, j: (i, j, 0)),
        pl.BlockSpec((DY, DZ), lambda i, j: (0, 0)),
        pl.BlockSpec((DY, 1), lambda i, j: (0, 0)),
    ]
    aliases = {}
    if out_prev is not None:
        args.append(out_prev)
        in_specs.append(pl.BlockSpec(memory_space=pl.ANY))
        aliases = {3: 0}

    def body(zt_ref, wt_ref, b_ref, *rest):
        _mm_body(zt_ref, wt_ref, b_ref, rest[-1])

    return pl.pallas_call(
        body,
        grid=(MS, NT // BN),
        in_specs=in_specs,
        out_specs=pl.BlockSpec(
            (1, DY, BN), lambda i, j: (slice_idx * MS + i, 0, j)
        ),
        out_shape=jax.ShapeDtypeStruct((M, DY, NT), jnp.float32),
        input_output_aliases=aliases,
    )(*args)


@jax.jit
def kernel(z_grid, mt, W, b):
    mt32 = mt.astype(jnp.int32)
    Wt = W.T
    b2 = b.reshape(DY, 1)
    zts = [_gather_fns[s](z_grid, mt32) for s in range(NSLICE)]
    out_t = None
    for s in range(NSLICE):
        out_t = _tc_linear_slice(s, zts[s], Wt, b2, out_t)
    return jnp.transpose(out_t, (0, 2, 1))


# speedup vs baseline: 2.6087x; 1.0417x over previous
import functools

import jax
import jax.numpy as jnp
from jax import lax
from jax.experimental import pallas as pl
from jax.experimental.pallas import tpu as pltpu
from jax.experimental.pallas import tpu_sc as plsc

M, G, DZ = 16, 16384, 128
NT, DY = 8192, 64
B = M * NT

NC, NS = 2, 16
NW = NC * NS
CH = 128
NBUF = 2

NSLICE = 4
MS = M // NSLICE
B_PER_W = MS * NT // NW
W_PER_ROW = NT // B_PER_W
NCHUNK = B_PER_W // CH


def _sc_gather(slice_idx):
    mesh = plsc.VectorSubcoreMesh(core_axis_name="c", subcore_axis_name="s")

    @functools.partial(
        pl.kernel,
        mesh=mesh,
        out_type=jax.ShapeDtypeStruct((MS, NT, DZ), jnp.float32),
        scratch_types=[
            pltpu.VMEM((B_PER_W,), jnp.int32),
            *[pltpu.VMEM((CH, DZ), jnp.float32) for _ in range(NBUF)],
            *[pltpu.SemaphoreType.DMA for _ in range(2 * NBUF)],
        ],
    )
    def gather(table_hbm, mt_hbm, out_hbm, idx_v, *bufs_and_sems):
        rows = bufs_and_sems[:NBUF]
        gsem = bufs_and_sems[NBUF : 2 * NBUF]
        ssem = bufs_and_sems[2 * NBUF :]
        wid = lax.axis_index("s") * NC + lax.axis_index("c")
        iloc = wid // W_PER_ROW
        i = slice_idx * MS + iloc
        h = wid % W_PER_ROW
        col0 = h * B_PER_W
        pltpu.sync_copy(mt_hbm.at[i, pl.ds(col0, B_PER_W)], idx_v)

        def gather_chunk(c, b):
            return pltpu.async_copy(
                table_hbm.at[i].at[idx_v.at[pl.ds(c * CH, CH)]], rows[b], gsem[b]
            )

        gcp = [None] * NBUF
        scp = [None] * NBUF
        gcp[0] = gather_chunk(0, 0)
        for c in range(NCHUNK):
            b = c % NBUF
            nb = (c + 1) % NBUF
            if c + 1 < NCHUNK:
                if scp[nb] is not None:
                    scp[nb].wait()
                gcp[nb] = gather_chunk(c + 1, nb)
            gcp[b].wait()
            scp[b] = pltpu.async_copy(
                rows[b], out_hbm.at[iloc, pl.ds(col0 + c * CH, CH)], ssem[b]
            )
        for b in range(NBUF):
            if scp[b] is not None:
                scp[b].wait()

    return gather


_gather_fns = [_sc_gather(s) for s in range(NSLICE)]


def _mm_body(zt_ref, wt_ref, b_ref, o_ref):
    o_ref[0] = (
        lax.dot_general(
            wt_ref[...],
            zt_ref[0],
            (((1,), (1,)), ((), ())),
            preferred_element_type=jnp.float32,
        )
        + b_ref[...]
    )


def _tc_linear_slice(slice_idx, zt_s, Wt, b2, out_prev):
    BN = 8192
    args = [zt_s, Wt, b2]
    in_specs = [
        pl.BlockSpec((1, BN, DZ), lambda i, j: (i, j, 0)),
        pl.BlockSpec((DY, DZ), lambda i, j: (0, 0)),
        pl.BlockSpec((DY, 1), lambda i, j: (0, 0)),
    ]
    aliases = {}
    if out_prev is not None:
        args.append(out_prev)
        in_specs.append(pl.BlockSpec(memory_space=pl.ANY))
        aliases = {3: 0}

    def body(zt_ref, wt_ref, b_ref, *rest):
        _mm_body(zt_ref, wt_ref, b_ref, rest[-1])

    return pl.pallas_call(
        body,
        grid=(MS, NT // BN),
        in_specs=in_specs,
        out_specs=pl.BlockSpec(
            (1, DY, BN), lambda i, j: (slice_idx * MS + i, 0, j)
        ),
        out_shape=jax.ShapeDtypeStruct((M, DY, NT), jnp.float32),
        input_output_aliases=aliases,
    )(*args)


@jax.jit
def kernel(z_grid, mt, W, b):
    mt32 = mt.astype(jnp.int32)
    Wt = W.T
    b2 = b.reshape(DY, 1)
    zts = [_gather_fns[s](z_grid, mt32) for s in range(NSLICE)]
    out_t = None
    for s in range(NSLICE):
        out_t = _tc_linear_slice(s, zts[s], Wt, b2, out_t)
    return jnp.transpose(out_t, (0, 2, 1))


# speedup vs baseline: 2.6451x; 1.0140x over previous
import functools

import jax
import jax.numpy as jnp
from jax import lax
from jax.experimental import pallas as pl
from jax.experimental.pallas import tpu as pltpu
from jax.experimental.pallas import tpu_sc as plsc

M, G, DZ = 16, 16384, 128
NT, DY = 8192, 64
B = M * NT

NC, NS = 2, 16
NW = NC * NS
CH = 128
NBUF = 4

NSLICE = 4
MS = M // NSLICE
B_PER_W = MS * NT // NW
W_PER_ROW = NT // B_PER_W
NCHUNK = B_PER_W // CH


def _sc_gather(slice_idx):
    mesh = plsc.VectorSubcoreMesh(core_axis_name="c", subcore_axis_name="s")

    @functools.partial(
        pl.kernel,
        mesh=mesh,
        out_type=jax.ShapeDtypeStruct((MS, NT, DZ), jnp.float32),
        scratch_types=[
            pltpu.VMEM((B_PER_W,), jnp.int32),
            *[pltpu.VMEM((CH, DZ), jnp.float32) for _ in range(NBUF)],
            *[pltpu.SemaphoreType.DMA for _ in range(2 * NBUF)],
        ],
    )
    def gather(table_hbm, mt_hbm, out_hbm, idx_v, *bufs_and_sems):
        rows = bufs_and_sems[:NBUF]
        gsem = bufs_and_sems[NBUF : 2 * NBUF]
        ssem = bufs_and_sems[2 * NBUF :]
        wid = lax.axis_index("s") * NC + lax.axis_index("c")
        iloc = wid // W_PER_ROW
        i = slice_idx * MS + iloc
        h = wid % W_PER_ROW
        col0 = h * B_PER_W
        pltpu.sync_copy(mt_hbm.at[i, pl.ds(col0, B_PER_W)], idx_v)

        def gather_chunk(c, b):
            return pltpu.async_copy(
                table_hbm.at[i].at[idx_v.at[pl.ds(c * CH, CH)]], rows[b], gsem[b]
            )

        gcp = [None] * NBUF
        scp = [None] * NBUF
        gcp[0] = gather_chunk(0, 0)
        for c in range(NCHUNK):
            b = c % NBUF
            nb = (c + 1) % NBUF
            if c + 1 < NCHUNK:
                if scp[nb] is not None:
                    scp[nb].wait()
                gcp[nb] = gather_chunk(c + 1, nb)
            gcp[b].wait()
            scp[b] = pltpu.async_copy(
                rows[b], out_hbm.at[iloc, pl.ds(col0 + c * CH, CH)], ssem[b]
            )
        for b in range(NBUF):
            if scp[b] is not None:
                scp[b].wait()

    return gather


_gather_fns = [_sc_gather(s) for s in range(NSLICE)]


def _mm_body(zt_ref, wt_ref, b_ref, o_ref):
    o_ref[0] = (
        lax.dot_general(
            wt_ref[...],
            zt_ref[0],
            (((1,), (1,)), ((), ())),
            preferred_element_type=jnp.float32,
        )
        + b_ref[...]
    )


def _tc_linear_slice(slice_idx, zt_s, Wt, b2, out_prev):
    BN = 8192
    args = [zt_s, Wt, b2]
    in_specs = [
        pl.BlockSpec((1, BN, DZ), lambda i, j: (i, j, 0)),
        pl.BlockSpec((DY, DZ), lambda i, j: (0, 0)),
        pl.BlockSpec((DY, 1), lambda i, j: (0, 0)),
    ]
    aliases = {}
    if out_prev is not None:
        args.append(out_prev)
        in_specs.append(pl.BlockSpec(memory_space=pl.ANY))
        aliases = {3: 0}

    def body(zt_ref, wt_ref, b_ref, *rest):
        _mm_body(zt_ref, wt_ref, b_ref, rest[-1])

    return pl.pallas_call(
        body,
        grid=(MS, NT // BN),
        in_specs=in_specs,
        out_specs=pl.BlockSpec(
            (1, DY, BN), lambda i, j: (slice_idx * MS + i, 0, j)
        ),
        out_shape=jax.ShapeDtypeStruct((M, DY, NT), jnp.float32),
        input_output_aliases=aliases,
    )(*args)


@jax.jit
def kernel(z_grid, mt, W, b):
    mt32 = mt.astype(jnp.int32)
    Wt = W.T
    b2 = b.reshape(DY, 1)
    zts = [_gather_fns[s](z_grid, mt32) for s in range(NSLICE)]
    out_t = None
    for s in range(NSLICE):
        out_t = _tc_linear_slice(s, zts[s], Wt, b2, out_t)
    return jnp.transpose(out_t, (0, 2, 1))
